# baseline (speedup 1.0000x reference)
import jax
import jax.numpy as jnp
from jax import lax
from jax.experimental import pallas as pl
from jax.experimental.pallas import tpu as pltpu
from jax.experimental.pallas import tpu_sc as plsc

L = 16
NW = 32
N = 21824
NG = 200
GP = 208
GF = 4 * GP
B = 224
PER_TILE = 688
SENT = 2.0e9

CHUNKS = ((0, 512, 512, 0), (16384, 128, 128, 512),
          (20480, 32, 32, 640), (21504, 16, 16, 672))


def _sc_body(anchor_h, gts_h, out_h,
             av, gv, bcx, bcy, bgi, outv,
             sem_g, sem_a0, sem_a1, sem_a2, sem_a3):
    nc = 2
    wid = lax.axis_index("s") * nc + lax.axis_index("c")
    asems = (sem_a0, sem_a1, sem_a2, sem_a3)

    cp_g = pltpu.async_copy(gts_h, gv, sem_g)
    copies = []
    for (base, step, cnt, voff), sem in zip(CHUNKS[:3], asems[:3]):
        start = pl.multiple_of(4 * (base + step * wid), 64)
        copies.append(pltpu.async_copy(
            anchor_h.at[pl.ds(start, 4 * cnt)], av.at[pl.ds(4 * voff, 4 * cnt)], sem))
    small = wid < 20

    @pl.when(small)
    def _():
        start = pl.multiple_of(4 * (21504 + 16 * wid), 64)
        pltpu.async_copy(anchor_h.at[pl.ds(start, 64)],
                         av.at[pl.ds(4 * 672, 64)], sem_a3).wait()

    cp_g.wait()

    sent_vec = jnp.full((L,), SENT, jnp.float32)
    for k in range(5 * B // L):
        bcx[pl.ds(k * L, L)] = sent_vec

    iota = lax.iota(jnp.int32, L)
    iota4 = iota * 4

    cnts = [jnp.zeros((L,), jnp.int32) for _ in range(5)]
    for k in range(GP // L):
        x0 = plsc.load_gather(gv, [iota4 + (64 * k + 0)])
        y0 = plsc.load_gather(gv, [iota4 + (64 * k + 1)])
        x1 = plsc.load_gather(gv, [iota4 + (64 * k + 2)])
        y1 = plsc.load_gather(gv, [iota4 + (64 * k + 3)])
        area = (x1 - x0) * (y1 - y0)
        lv = jnp.zeros((L,), jnp.float32)
        for thr in (1024.0, 4096.0, 16384.0, 65536.0):
            lv = lv + jnp.where(area >= thr, 1.0, 0.0).astype(jnp.float32)
        lv = jnp.where(area >= 262144.0, 0.0, lv)
        cx = (x0 + x1) * 0.5
        cy = (y0 + y1) * 0.5
        gidx = iota + 16 * k
        valid = gidx < NG
        for l in range(5):
            m = (lv == float(l)) & valid
            r = plsc.cumsum(m.astype(jnp.int32))
            dest = cnts[l] + r + (B * l - 1)
            plsc.store_scatter(bcx, [dest], cx, mask=m)
            plsc.store_scatter(bcy, [dest], cy, mask=m)
            plsc.store_scatter(bgi, [dest], gidx, mask=m)
            cnts[l] = cnts[l] + plsc.all_reduce_population_count(m)
    c = [jnp.max(cnts[l]) for l in range(5)]

    for handle in copies:
        handle.wait()

    neg2 = jnp.full((L,), -2, jnp.int32)

    def scan_group(slots, base, n, unroll):
        boxes = []
        for i in slots:
            boxes.append([plsc.load_gather(av, [iota4 + (64 * i + cc)])
                          for cc in range(4)])
        nq = (n + (unroll - 1)) // unroll

        def body(q, assigns):
            out = list(assigns)
            j = base + q * unroll
            for u in range(unroll):
                idx = jnp.full((L,), j + u, jnp.int32)
                bx = plsc.load_gather(bcx, [idx])
                by = plsc.load_gather(bcy, [idx])
                bg = plsc.load_gather(bgi, [idx])
                for si, (a0, a1, a2, a3) in enumerate(boxes):
                    m = (bx > a0) & (by > a1) & (bx < a2) & (by < a3)
                    out[si] = jnp.where(m, bg, out[si])
            return tuple(out)

        assigns = lax.fori_loop(0, nq, body, tuple(neg2 for _ in slots))
        for si, i in enumerate(slots):
            outv[pl.ds(16 * i, L)] = assigns[si]

    for g0 in range(0, 32, 4):
        scan_group(list(range(g0, g0 + 4)), 0 * B, c[0], 2)
    scan_group([32, 33, 34, 35], 1 * B, c[1], 2)
    scan_group([36, 37, 38, 39], 1 * B, c[1], 2)
    scan_group([40, 41], 2 * B, c[2], 2)

    @pl.when(small)
    def _():
        is3 = wid < 16
        base = jnp.where(is3, 3 * B, 4 * B)
        n = jnp.where(is3, c[3], c[4])
        scan_group([42], base, n, 4)

    pltpu.sync_copy(outv.at[pl.ds(0, 512)],
                    out_h.at[pl.ds(pl.multiple_of(512 * wid, 16), 512)])
    pltpu.sync_copy(outv.at[pl.ds(512, 128)],
                    out_h.at[pl.ds(pl.multiple_of(16384 + 128 * wid, 16), 128)])
    pltpu.sync_copy(outv.at[pl.ds(640, 32)],
                    out_h.at[pl.ds(pl.multiple_of(20480 + 32 * wid, 16), 32)])

    @pl.when(small)
    def _():
        pltpu.sync_copy(outv.at[pl.ds(672, 16)],
                        out_h.at[pl.ds(pl.multiple_of(21504 + 16 * wid, 16), 16)])


@jax.jit
def kernel(anchor, gts):
    aflat = anchor.reshape(-1)
    gflat = jnp.full((GF,), SENT, jnp.float32).at[: 4 * gts.shape[0]].set(
        gts.reshape(-1))

    mesh = plsc.VectorSubcoreMesh(core_axis_name="c", subcore_axis_name="s")
    run = pl.kernel(
        _sc_body,
        mesh=mesh,
        compiler_params=pltpu.CompilerParams(needs_layout_passes=False,
                                             skip_device_barrier=True),
        out_type=jax.ShapeDtypeStruct((N,), jnp.int32),
        scratch_types=[
            pltpu.VMEM((4 * PER_TILE,), jnp.float32),
            pltpu.VMEM((GF,), jnp.float32),
            pltpu.VMEM((5 * B,), jnp.float32),
            pltpu.VMEM((5 * B,), jnp.float32),
            pltpu.VMEM((5 * B,), jnp.int32),
            pltpu.VMEM((PER_TILE,), jnp.int32),
            pltpu.SemaphoreType.DMA,
            pltpu.SemaphoreType.DMA,
            pltpu.SemaphoreType.DMA,
            pltpu.SemaphoreType.DMA,
            pltpu.SemaphoreType.DMA,
        ],
    )
    return run(aflat, gflat).astype(jnp.int64)


# speedup vs baseline: 1.1697x; 1.1697x over previous
import jax
import jax.numpy as jnp
from jax import lax
from jax.experimental import pallas as pl
from jax.experimental.pallas import tpu as pltpu
from jax.experimental.pallas import tpu_sc as plsc

L = 16
NW = 32
N = 21824
NG = 200
GP = 208
GF = 4 * GP
B = 224
PER_TILE = 688
SENT = 2.0e9

CHUNKS = ((0, 512, 512, 0), (16384, 128, 128, 512),
          (20480, 32, 32, 640), (21504, 16, 16, 672))


def _sc_body(anchor_h, gts_h, out_h,
             av, gv, bcx, bcy, bgi, outv,
             sem_g, sem_a0, sem_a1, sem_a2, sem_a3):
    nc = 2
    wid = lax.axis_index("s") * nc + lax.axis_index("c")
    asems = (sem_a0, sem_a1, sem_a2, sem_a3)

    cp_g = pltpu.async_copy(gts_h, gv, sem_g)
    copies = []
    for (base, step, cnt, voff), sem in zip(CHUNKS[:3], asems[:3]):
        start = pl.multiple_of(base + step * wid, 16)
        copies.append(pltpu.async_copy(
            anchor_h.at[pl.ds(start, cnt)], av.at[pl.ds(voff, cnt)], sem))
    small = wid < 20

    @pl.when(small)
    def _():
        start = pl.multiple_of(21504 + 16 * wid, 16)
        pltpu.async_copy(anchor_h.at[pl.ds(start, 16)],
                         av.at[pl.ds(672, 16)], sem_a3).wait()

    sent_vec = jnp.full((L,), SENT, jnp.float32)
    for k in range(5 * B // L):
        bcx[pl.ds(k * L, L)] = sent_vec

    iota = lax.iota(jnp.int32, L)
    iota4 = iota * 4

    cp_g.wait()

    cnts = [jnp.zeros((L,), jnp.int32) for _ in range(5)]
    for k in range(GP // L):
        x0 = plsc.load_gather(gv, [iota4 + (64 * k + 0)])
        y0 = plsc.load_gather(gv, [iota4 + (64 * k + 1)])
        x1 = plsc.load_gather(gv, [iota4 + (64 * k + 2)])
        y1 = plsc.load_gather(gv, [iota4 + (64 * k + 3)])
        area = (x1 - x0) * (y1 - y0)
        lv = jnp.zeros((L,), jnp.float32)
        for thr in (1024.0, 4096.0, 16384.0, 65536.0):
            lv = lv + jnp.where(area >= thr, 1.0, 0.0).astype(jnp.float32)
        lv = jnp.where(area >= 262144.0, 0.0, lv)
        cx = (x0 + x1) * 0.5
        cy = (y0 + y1) * 0.5
        gidx = iota + 16 * k
        valid = gidx < NG
        for l in range(5):
            m = (lv == float(l)) & valid
            r = plsc.cumsum(m.astype(jnp.int32))
            dest = cnts[l] + r + (B * l - 1)
            plsc.store_scatter(bcx, [dest], cx, mask=m)
            plsc.store_scatter(bcy, [dest], cy, mask=m)
            plsc.store_scatter(bgi, [dest], gidx, mask=m)
            cnts[l] = cnts[l] + plsc.all_reduce_population_count(m)
    c = [jnp.max(cnts[l]) for l in range(5)]

    for handle in copies:
        handle.wait()

    neg2 = jnp.full((L,), -2, jnp.int32)

    def scan_group(slots, base, n, unroll):
        @pl.when(n == 0)
        def _():
            for i in slots:
                outv[pl.ds(16 * i, L)] = neg2

        @pl.when(n > 0)
        def _():
            boxes = []
            for i in slots:
                row = iota + 16 * i
                boxes.append([plsc.load_gather(av, [row, jnp.full((L,), cc, jnp.int32)])
                              for cc in range(4)])
            nq = (n + (unroll - 1)) // unroll

            def body(q, assigns):
                out = list(assigns)
                j = base + q * unroll
                for u in range(unroll):
                    idx = jnp.full((L,), j + u, jnp.int32)
                    bx = plsc.load_gather(bcx, [idx])
                    by = plsc.load_gather(bcy, [idx])
                    bg = plsc.load_gather(bgi, [idx])
                    for si, (a0, a1, a2, a3) in enumerate(boxes):
                        m = (bx > a0) & (by > a1) & (bx < a2) & (by < a3)
                        out[si] = jnp.where(m, bg, out[si])
                return tuple(out)

            assigns = lax.fori_loop(0, nq, body, tuple(neg2 for _ in slots))
            for si, i in enumerate(slots):
                outv[pl.ds(16 * i, L)] = assigns[si]

    for g0 in range(0, 32, 4):
        scan_group(list(range(g0, g0 + 4)), 0 * B, c[0], 4)
    scan_group([32, 33, 34, 35], 1 * B, c[1], 4)
    scan_group([36, 37, 38, 39], 1 * B, c[1], 4)
    scan_group([40, 41], 2 * B, c[2], 4)

    @pl.when(small)
    def _():
        is3 = wid < 16
        base = jnp.where(is3, 3 * B, 4 * B)
        n = jnp.where(is3, c[3], c[4])
        scan_group([42], base, n, 4)

    o0 = pltpu.async_copy(outv.at[pl.ds(0, 512)],
                          out_h.at[pl.ds(pl.multiple_of(512 * wid, 16), 512)],
                          sem_a0)
    o1 = pltpu.async_copy(outv.at[pl.ds(512, 128)],
                          out_h.at[pl.ds(pl.multiple_of(16384 + 128 * wid, 16), 128)],
                          sem_a1)
    o2 = pltpu.async_copy(outv.at[pl.ds(640, 32)],
                          out_h.at[pl.ds(pl.multiple_of(20480 + 32 * wid, 16), 32)],
                          sem_a2)

    @pl.when(small)
    def _():
        pltpu.async_copy(outv.at[pl.ds(672, 16)],
                         out_h.at[pl.ds(pl.multiple_of(21504 + 16 * wid, 16), 16)],
                         sem_a3).wait()

    o0.wait()
    o1.wait()
    o2.wait()


@jax.jit
def kernel(anchor, gts):
    gflat = jnp.full((GF,), SENT, jnp.float32).at[: 4 * gts.shape[0]].set(
        gts.reshape(-1))

    mesh = plsc.VectorSubcoreMesh(core_axis_name="c", subcore_axis_name="s")
    run = pl.kernel(
        _sc_body,
        mesh=mesh,
        compiler_params=pltpu.CompilerParams(needs_layout_passes=False,
                                             skip_device_barrier=True),
        out_type=jax.ShapeDtypeStruct((N,), jnp.int32),
        scratch_types=[
            pltpu.VMEM((PER_TILE, 4), jnp.float32),
            pltpu.VMEM((GF,), jnp.float32),
            pltpu.VMEM((5 * B,), jnp.float32),
            pltpu.VMEM((5 * B,), jnp.float32),
            pltpu.VMEM((5 * B,), jnp.int32),
            pltpu.VMEM((PER_TILE,), jnp.int32),
            pltpu.SemaphoreType.DMA,
            pltpu.SemaphoreType.DMA,
            pltpu.SemaphoreType.DMA,
            pltpu.SemaphoreType.DMA,
            pltpu.SemaphoreType.DMA,
        ],
    )
    return run(anchor, gflat).astype(jnp.int64)


# speedup vs baseline: 1.3629x; 1.1652x over previous
import jax
import jax.numpy as jnp
from jax import lax
from jax.experimental import pallas as pl
from jax.experimental.pallas import tpu as pltpu
from jax.experimental.pallas import tpu_sc as plsc

L = 16
NW = 32
N = 21824
NG = 200
GP = 208
B = 224
PER_TILE = 688
SENT = 2.0e9
NGRP = 12

CHUNKS = ((0, 512, 512, 0), (16384, 128, 128, 512),
          (20480, 32, 32, 640), (21504, 16, 16, 672))


def _sc_body(a32_h, gts_h, out_h,
             av, gv, bcx, bcy, bgi, outv, tbl,
             sem_g, sem_a0, sem_a1, sem_a2, sem_a3):
    nc = 2
    wid = lax.axis_index("s") * nc + lax.axis_index("c")

    cp_g = pltpu.async_copy(gts_h, gv, sem_g)
    copies = [pltpu.async_copy(a32_h.at[wid], av, sem_a0)]
    small = wid < 20

    sent_vec = jnp.full((L,), SENT, jnp.float32)
    for k in range(5 * B // L):
        bcx[pl.ds(k * L, L)] = sent_vec

    iota = lax.iota(jnp.int32, L)
    cp_g.wait()

    def prep(k, cnts):
        o = k * L
        x0 = gv[pl.ds(0 * GP + o, L)]
        y0 = gv[pl.ds(1 * GP + o, L)]
        x1 = gv[pl.ds(2 * GP + o, L)]
        y1 = gv[pl.ds(3 * GP + o, L)]
        area = (x1 - x0) * (y1 - y0)
        lv = jnp.zeros((L,), jnp.float32)
        for thr in (1024.0, 4096.0, 16384.0, 65536.0):
            lv = lv + jnp.where(area >= thr, 1.0, 0.0).astype(jnp.float32)
        lv = jnp.where(area >= 262144.0, 0.0, lv)
        cx = (x0 + x1) * 0.5
        cy = (y0 + y1) * 0.5
        gidx = iota + o
        valid = gidx < NG
        out = []
        for l in range(5):
            m = (lv == float(l)) & valid
            r = plsc.cumsum(m.astype(jnp.int32))
            dest = cnts[l] + r + (B * l - 1)
            plsc.store_scatter(bcx, [dest], cx, mask=m)
            plsc.store_scatter(bcy, [dest], cy, mask=m)
            plsc.store_scatter(bgi, [dest], gidx, mask=m)
            out.append(cnts[l] + plsc.all_reduce_population_count(m))
        return tuple(out)

    cnts = lax.fori_loop(0, GP // L, prep,
                         tuple(jnp.zeros((L,), jnp.int32) for _ in range(5)))
    c = [jnp.max(cnts[l]) for l in range(5)]

    for g in range(8):
        for u in range(4):
            tbl[6 * g + u] = 4 * g + u
        tbl[6 * g + 4] = 0
        tbl[6 * g + 5] = c[0]
    for g, slots, bb, n in ((8, (32, 33, 34, 35), B, c[1]),
                            (9, (36, 37, 38, 39), B, c[1]),
                            (10, (40, 41, 40, 41), 2 * B, c[2])):
        for u in range(4):
            tbl[6 * g + u] = slots[u]
        tbl[6 * g + 4] = bb
        tbl[6 * g + 5] = n
    for u in range(4):
        tbl[6 * 11 + u] = 42
    tbl[6 * 11 + 4] = jnp.where(wid < 16, 3 * B, 4 * B)
    tbl[6 * 11 + 5] = jnp.where(small, jnp.where(wid < 16, c[3], c[4]), 0)

    for handle in copies:
        handle.wait()

    neg2 = jnp.full((L,), -2, jnp.int32)

    def group(g, carry):
        t0 = 6 * g
        rows = [iota + 16 * tbl[t0 + u] for u in range(4)]
        bb = tbl[t0 + 4]
        n = tbl[t0 + 5]

        @pl.when(n == 0)
        def _():
            for u in range(4):
                plsc.store_scatter(outv, [rows[u]], neg2)

        @pl.when(n > 0)
        def _():
            boxes = []
            for u in range(4):
                r4 = rows[u] * 4
                boxes.append([plsc.load_gather(av, [r4 + cc])
                              for cc in range(4)])

            def body(q, assigns):
                out = list(assigns)
                j = bb + q * 4
                for u in range(4):
                    idx = jnp.full((L,), j + u, jnp.int32)
                    bxv = plsc.load_gather(bcx, [idx])
                    byv = plsc.load_gather(bcy, [idx])
                    bgv = plsc.load_gather(bgi, [idx])
                    for si, (a0, a1, a2, a3) in enumerate(boxes):
                        m = (bxv > a0) & (byv > a1) & (bxv < a2) & (byv < a3)
                        out[si] = jnp.where(m, bgv, out[si])
                return tuple(out)

            nq = (n + 3) >> 2
            assigns = lax.fori_loop(0, nq, body, (neg2, neg2, neg2, neg2))
            for u in range(4):
                plsc.store_scatter(outv, [rows[u]], assigns[u])
        return carry

    lax.fori_loop(0, NGRP, group, 0)

    o0 = pltpu.async_copy(outv.at[pl.ds(0, 512)],
                          out_h.at[pl.ds(pl.multiple_of(512 * wid, 16), 512)],
                          sem_a0)
    o1 = pltpu.async_copy(outv.at[pl.ds(512, 128)],
                          out_h.at[pl.ds(pl.multiple_of(16384 + 128 * wid, 16), 128)],
                          sem_a1)
    o2 = pltpu.async_copy(outv.at[pl.ds(640, 32)],
                          out_h.at[pl.ds(pl.multiple_of(20480 + 32 * wid, 16), 32)],
                          sem_a2)

    @pl.when(small)
    def _():
        pltpu.async_copy(outv.at[pl.ds(672, 16)],
                         out_h.at[pl.ds(pl.multiple_of(21504 + 16 * wid, 16), 16)],
                         sem_a3).wait()

    o0.wait()
    o1.wait()
    o2.wait()


@jax.jit
def kernel(anchor, gts):
    c0 = anchor[:16384].reshape(NW, 512, 4)
    c1 = anchor[16384:20480].reshape(NW, 128, 4)
    c2 = anchor[20480:21504].reshape(NW, 32, 4)
    c34 = jnp.concatenate([anchor[21504:21824].reshape(20, 16, 4),
                           jnp.zeros((12, 16, 4), jnp.float32)], axis=0)
    a32 = jnp.concatenate([c0, c1, c2, c34], axis=1).reshape(NW, 4 * PER_TILE)
    gcols = jnp.full((4, GP), SENT, jnp.float32).at[:, :NG].set(gts.T)
    gflat = gcols.reshape(-1)

    mesh = plsc.VectorSubcoreMesh(core_axis_name="c", subcore_axis_name="s")
    run = pl.kernel(
        _sc_body,
        mesh=mesh,
        compiler_params=pltpu.CompilerParams(needs_layout_passes=False,
                                             skip_device_barrier=True),
        out_type=jax.ShapeDtypeStruct((N,), jnp.int32),
        scratch_types=[
            pltpu.VMEM((4 * PER_TILE,), jnp.float32),
            pltpu.VMEM((4 * GP,), jnp.float32),
            pltpu.VMEM((5 * B,), jnp.float32),
            pltpu.VMEM((5 * B,), jnp.float32),
            pltpu.VMEM((5 * B,), jnp.int32),
            pltpu.VMEM((PER_TILE,), jnp.int32),
            pltpu.SMEM((6 * NGRP,), jnp.int32),
            pltpu.SemaphoreType.DMA,
            pltpu.SemaphoreType.DMA,
            pltpu.SemaphoreType.DMA,
            pltpu.SemaphoreType.DMA,
            pltpu.SemaphoreType.DMA,
        ],
    )
    return run(a32, gflat).astype(jnp.int64)


# speedup vs baseline: 1.5192x; 1.1147x over previous
import jax
import jax.numpy as jnp
from jax import lax
from jax.experimental import pallas as pl
from jax.experimental.pallas import tpu as pltpu
from jax.experimental.pallas import tpu_sc as plsc

L = 16
NW = 32
N = 21824
NG = 200
GP = 208
B = 224
PER_TILE = 688
SENT = 2.0e9
NGRP = 12

CHUNKS = ((0, 512, 512, 0), (16384, 128, 128, 512),
          (20480, 32, 32, 640), (21504, 16, 16, 672))


def _sc_body(ax_h, ay_h, bx_h, by_h, gts_h, out_h,
             av, gv, bcx, bcy, bgi, outv, tbl,
             sem_g, sem_a0, sem_a1, sem_a2, sem_a3):
    nc = 2
    wid = lax.axis_index("s") * nc + lax.axis_index("c")
    asems = (sem_a0, sem_a1, sem_a2, sem_a3)
    comps = (ax_h, ay_h, bx_h, by_h)

    cp_g = pltpu.async_copy(gts_h, gv, sem_g)
    copies = []
    for (base, step, cnt, voff), sem in zip(CHUNKS[:3], asems[:3]):
        start = pl.multiple_of(base + step * wid, 16)
        for cc in range(4):
            copies.append(pltpu.async_copy(
                comps[cc].at[pl.ds(start, cnt)],
                av.at[pl.ds(688 * cc + voff, cnt)], sem))
    small = wid < 20

    @pl.when(small)
    def _():
        start = pl.multiple_of(21504 + 16 * wid, 16)
        hs = [pltpu.async_copy(comps[cc].at[pl.ds(start, 16)],
                               av.at[pl.ds(688 * cc + 672, 16)], sem_a3)
              for cc in range(4)]
        for h in hs:
            h.wait()

    sent_vec = jnp.full((L,), SENT, jnp.float32)
    iota = lax.iota(jnp.int32, L)
    cp_g.wait()

    def prep(k, cnts):
        o = k * L
        x0 = gv[pl.ds(0 * GP + o, L)]
        y0 = gv[pl.ds(1 * GP + o, L)]
        x1 = gv[pl.ds(2 * GP + o, L)]
        y1 = gv[pl.ds(3 * GP + o, L)]
        area = (x1 - x0) * (y1 - y0)
        lv = jnp.zeros((L,), jnp.float32)
        for thr in (1024.0, 4096.0, 16384.0, 65536.0):
            lv = lv + jnp.where(area >= thr, 1.0, 0.0).astype(jnp.float32)
        lv = jnp.where(area >= 262144.0, 0.0, lv)
        cx = (x0 + x1) * 0.5
        cy = (y0 + y1) * 0.5
        gidx = iota + o
        valid = gidx < NG
        out = []
        for l in range(5):
            m = (lv == float(l)) & valid
            r = plsc.cumsum(m.astype(jnp.int32))
            dest = cnts[l] + r + (B * l - 1)
            plsc.store_scatter(bcx, [dest], cx, mask=m)
            plsc.store_scatter(bcy, [dest], cy, mask=m)
            plsc.store_scatter(bgi, [dest], gidx, mask=m)
            out.append(cnts[l] + plsc.all_reduce_population_count(m))
        return tuple(out)

    cnts = lax.fori_loop(0, GP // L, prep,
                         tuple(jnp.zeros((L,), jnp.int32) for _ in range(5)))
    c = [jnp.max(cnts[l]) for l in range(5)]
    for l in range(5):
        plsc.store_scatter(bcx, [iota + (B * l + c[l])], sent_vec)

    for g in range(8):
        for u in range(4):
            tbl[6 * g + u] = 4 * g + u
        tbl[6 * g + 4] = 0
        tbl[6 * g + 5] = c[0]
    for g, slots, bb, n in ((8, (32, 33, 34, 35), B, c[1]),
                            (9, (36, 37, 38, 39), B, c[1]),
                            (10, (40, 41, 40, 41), 2 * B, c[2])):
        for u in range(4):
            tbl[6 * g + u] = slots[u]
        tbl[6 * g + 4] = bb
        tbl[6 * g + 5] = n
    for u in range(4):
        tbl[6 * 11 + u] = 42
    tbl[6 * 11 + 4] = jnp.where(wid < 16, 3 * B, 4 * B)
    tbl[6 * 11 + 5] = jnp.where(small, jnp.where(wid < 16, c[3], c[4]), 0)

    for handle in copies:
        handle.wait()

    neg2 = jnp.full((L,), -2, jnp.int32)

    def group(g, carry):
        t0 = 6 * g
        rows = [iota + 16 * tbl[t0 + u] for u in range(4)]
        bb = tbl[t0 + 4]
        n = tbl[t0 + 5]

        @pl.when(n == 0)
        def _():
            for u in range(4):
                plsc.store_scatter(outv, [rows[u]], neg2)

        @pl.when(n > 0)
        def _():
            boxes = []
            for u in range(4):
                boxes.append([plsc.load_gather(av, [rows[u] + 688 * cc])
                              for cc in range(4)])

            def body(q, assigns):
                out = list(assigns)
                j = bb + q * 4
                for u in range(4):
                    idx = jnp.full((L,), j + u, jnp.int32)
                    bxv = plsc.load_gather(bcx, [idx])
                    byv = plsc.load_gather(bcy, [idx])
                    bgv = plsc.load_gather(bgi, [idx])
                    for si, (a0, a1, a2, a3) in enumerate(boxes):
                        m = (bxv > a0) & (byv > a1) & (bxv < a2) & (byv < a3)
                        out[si] = jnp.where(m, bgv, out[si])
                return tuple(out)

            nq = (n + 3) >> 2
            assigns = lax.fori_loop(0, nq, body, (neg2, neg2, neg2, neg2))
            for u in range(4):
                plsc.store_scatter(outv, [rows[u]], assigns[u])
        return carry

    lax.fori_loop(0, NGRP, group, 0)

    o0 = pltpu.async_copy(outv.at[pl.ds(0, 512)],
                          out_h.at[pl.ds(pl.multiple_of(512 * wid, 16), 512)],
                          sem_a0)
    o1 = pltpu.async_copy(outv.at[pl.ds(512, 128)],
                          out_h.at[pl.ds(pl.multiple_of(16384 + 128 * wid, 16), 128)],
                          sem_a1)
    o2 = pltpu.async_copy(outv.at[pl.ds(640, 32)],
                          out_h.at[pl.ds(pl.multiple_of(20480 + 32 * wid, 16), 32)],
                          sem_a2)

    @pl.when(small)
    def _():
        pltpu.async_copy(outv.at[pl.ds(672, 16)],
                         out_h.at[pl.ds(pl.multiple_of(21504 + 16 * wid, 16), 16)],
                         sem_a3).wait()

    o0.wait()
    o1.wait()
    o2.wait()


@jax.jit
def kernel(anchor, gts):
    cols = [anchor[:, cc] for cc in range(4)]
    gflat = jnp.pad(gts.T, ((0, 0), (0, GP - NG)),
                    constant_values=SENT).reshape(-1)

    mesh = plsc.VectorSubcoreMesh(core_axis_name="c", subcore_axis_name="s")
    run = pl.kernel(
        _sc_body,
        mesh=mesh,
        compiler_params=pltpu.CompilerParams(needs_layout_passes=False,
                                             skip_device_barrier=True),
        out_type=jax.ShapeDtypeStruct((N,), jnp.int32),
        scratch_types=[
            pltpu.VMEM((4 * PER_TILE,), jnp.float32),
            pltpu.VMEM((4 * GP,), jnp.float32),
            pltpu.VMEM((5 * B,), jnp.float32),
            pltpu.VMEM((5 * B,), jnp.float32),
            pltpu.VMEM((5 * B,), jnp.int32),
            pltpu.VMEM((PER_TILE,), jnp.int32),
            pltpu.SMEM((6 * NGRP,), jnp.int32),
            pltpu.SemaphoreType.DMA,
            pltpu.SemaphoreType.DMA,
            pltpu.SemaphoreType.DMA,
            pltpu.SemaphoreType.DMA,
            pltpu.SemaphoreType.DMA,
        ],
    )
    return run(*cols, gflat).astype(jnp.int64)


# speedup vs baseline: 1.5214x; 1.0014x over previous
import jax
import jax.numpy as jnp
from jax import lax
from jax.experimental import pallas as pl
from jax.experimental.pallas import tpu as pltpu
from jax.experimental.pallas import tpu_sc as plsc

L = 16
NW = 32
N = 21824
NG = 200
GP = 208
B = 224
PER_TILE = 688
SENT = 2.0e9
NGRP = 12

CHUNKS = ((0, 512, 512, 0), (16384, 128, 128, 512),
          (20480, 32, 32, 640), (21504, 16, 16, 672))


def _sc_body(ax_h, ay_h, bx_h, by_h, gts_h, out_h,
             av, gv, bcx, bcy, bgi, outv, tbl,
             sem_g, sem_a0, sem_a1, sem_a2, sem_a3):
    nc = 2
    wid = lax.axis_index("s") * nc + lax.axis_index("c")
    asems = (sem_a0, sem_a1, sem_a2, sem_a3)
    comps = (ax_h, ay_h, bx_h, by_h)

    cp_g = pltpu.async_copy(gts_h, gv, sem_g)
    copies = []
    for (base, step, cnt, voff), sem in zip(CHUNKS[:3], asems[:3]):
        start = pl.multiple_of(base + step * wid, 16)
        for cc in range(4):
            copies.append(pltpu.async_copy(
                comps[cc].at[pl.ds(start, cnt)],
                av.at[pl.ds(688 * cc + voff, cnt)], sem))
    small = wid < 20

    @pl.when(small)
    def _():
        start = pl.multiple_of(21504 + 16 * wid, 16)
        hs = [pltpu.async_copy(comps[cc].at[pl.ds(start, 16)],
                               av.at[pl.ds(688 * cc + 672, 16)], sem_a3)
              for cc in range(4)]
        for h in hs:
            h.wait()

    sent_vec = jnp.full((L,), SENT, jnp.float32)
    iota = lax.iota(jnp.int32, L)
    cp_g.wait()

    def prep(k, cnts):
        o = k * L
        x0 = gv[pl.ds(0 * GP + o, L)]
        y0 = gv[pl.ds(1 * GP + o, L)]
        x1 = gv[pl.ds(2 * GP + o, L)]
        y1 = gv[pl.ds(3 * GP + o, L)]
        area = (x1 - x0) * (y1 - y0)
        lv = jnp.zeros((L,), jnp.float32)
        for thr in (1024.0, 4096.0, 16384.0, 65536.0):
            lv = lv + jnp.where(area >= thr, 1.0, 0.0).astype(jnp.float32)
        lv = jnp.where(area >= 262144.0, 0.0, lv)
        cx = (x0 + x1) * 0.5
        cy = (y0 + y1) * 0.5
        gidx = iota + o
        valid = gidx < NG
        out = []
        for l in range(5):
            m = (lv == float(l)) & valid
            r = plsc.cumsum(m.astype(jnp.int32))
            dest = cnts[l] + r + (B * l - 1)
            plsc.store_scatter(bcx, [dest], cx, mask=m)
            plsc.store_scatter(bcy, [dest], cy, mask=m)
            plsc.store_scatter(bgi, [dest], gidx, mask=m)
            out.append(cnts[l] + plsc.all_reduce_population_count(m))
        return tuple(out)

    cnts = lax.fori_loop(0, GP // L, prep,
                         tuple(jnp.zeros((L,), jnp.int32) for _ in range(5)))
    c = [jnp.max(cnts[l]) for l in range(5)]
    for l in range(5):
        plsc.store_scatter(bcx, [iota + (B * l + c[l])], sent_vec)

    for g in range(8):
        for u in range(4):
            tbl[6 * g + u] = 4 * g + u
        tbl[6 * g + 4] = 0
        tbl[6 * g + 5] = c[0]
    for g, slots, bb, n in ((8, (32, 33, 34, 35), B, c[1]),
                            (9, (36, 37, 38, 39), B, c[1]),
                            (10, (40, 41, 40, 41), 2 * B, c[2])):
        for u in range(4):
            tbl[6 * g + u] = slots[u]
        tbl[6 * g + 4] = bb
        tbl[6 * g + 5] = n
    for u in range(4):
        tbl[6 * 11 + u] = 42
    tbl[6 * 11 + 4] = jnp.where(wid < 16, 3 * B, 4 * B)
    tbl[6 * 11 + 5] = jnp.where(small, jnp.where(wid < 16, c[3], c[4]), 0)

    for handle in copies:
        handle.wait()

    neg2 = jnp.full((L,), -2, jnp.int32)

    def group(g, carry):
        t0 = 6 * g
        rows = [iota + 16 * tbl[t0 + u] for u in range(4)]
        bb = tbl[t0 + 4]
        n = tbl[t0 + 5]

        @pl.when(n == 0)
        def _():
            for u in range(4):
                plsc.store_scatter(outv, [rows[u]], neg2)

        @pl.when(n > 0)
        def _():
            boxes = []
            for u in range(4):
                boxes.append([plsc.load_gather(av, [rows[u] + 688 * cc])
                              for cc in range(4)])

            nq = (n + 3) >> 2

            @plsc.parallel_loop(0, nq, carry=(neg2, neg2, neg2, neg2))
            def scan(q, assigns):
                out = list(assigns)
                j = bb + q * 4
                for u in range(4):
                    idx = jnp.full((L,), j + u, jnp.int32)
                    bxv = plsc.load_gather(bcx, [idx])
                    byv = plsc.load_gather(bcy, [idx])
                    bgv = plsc.load_gather(bgi, [idx])
                    for si, (a0, a1, a2, a3) in enumerate(boxes):
                        m = (bxv > a0) & (byv > a1) & (bxv < a2) & (byv < a3)
                        out[si] = jnp.where(m, bgv, out[si])
                return tuple(out)

            assigns = scan
            for u in range(4):
                plsc.store_scatter(outv, [rows[u]], assigns[u])
        return carry

    lax.fori_loop(0, NGRP, group, 0)

    o0 = pltpu.async_copy(outv.at[pl.ds(0, 512)],
                          out_h.at[pl.ds(pl.multiple_of(512 * wid, 16), 512)],
                          sem_a0)
    o1 = pltpu.async_copy(outv.at[pl.ds(512, 128)],
                          out_h.at[pl.ds(pl.multiple_of(16384 + 128 * wid, 16), 128)],
                          sem_a1)
    o2 = pltpu.async_copy(outv.at[pl.ds(640, 32)],
                          out_h.at[pl.ds(pl.multiple_of(20480 + 32 * wid, 16), 32)],
                          sem_a2)

    @pl.when(small)
    def _():
        pltpu.async_copy(outv.at[pl.ds(672, 16)],
                         out_h.at[pl.ds(pl.multiple_of(21504 + 16 * wid, 16), 16)],
                         sem_a3).wait()

    o0.wait()
    o1.wait()
    o2.wait()


@jax.jit
def kernel(anchor, gts):
    cols = [anchor[:, cc] for cc in range(4)]
    gflat = jnp.pad(gts.T, ((0, 0), (0, GP - NG)),
                    constant_values=SENT).reshape(-1)

    mesh = plsc.VectorSubcoreMesh(core_axis_name="c", subcore_axis_name="s")
    run = pl.kernel(
        _sc_body,
        mesh=mesh,
        compiler_params=pltpu.CompilerParams(needs_layout_passes=False,
                                             skip_device_barrier=True),
        out_type=jax.ShapeDtypeStruct((N,), jnp.int32),
        scratch_types=[
            pltpu.VMEM((4 * PER_TILE,), jnp.float32),
            pltpu.VMEM((4 * GP,), jnp.float32),
            pltpu.VMEM((5 * B,), jnp.float32),
            pltpu.VMEM((5 * B,), jnp.float32),
            pltpu.VMEM((5 * B,), jnp.int32),
            pltpu.VMEM((PER_TILE,), jnp.int32),
            pltpu.SMEM((6 * NGRP,), jnp.int32),
            pltpu.SemaphoreType.DMA,
            pltpu.SemaphoreType.DMA,
            pltpu.SemaphoreType.DMA,
            pltpu.SemaphoreType.DMA,
            pltpu.SemaphoreType.DMA,
        ],
    )
    return run(*cols, gflat).astype(jnp.int64)


# speedup vs baseline: 1.6313x; 1.0722x over previous
import jax
import jax.numpy as jnp
from jax import lax
from jax.experimental import pallas as pl
from jax.experimental.pallas import tpu as pltpu
from jax.experimental.pallas import tpu_sc as plsc

L = 16
NW = 32
N = 21824
NG = 200
GP = 208
B = 224
PER_TILE = 688
SENT = 2.0e9
NGRP = 12

CHUNKS = ((0, 512, 512, 0), (16384, 128, 128, 512),
          (20480, 32, 32, 640), (21504, 16, 16, 672))


def _sc_body(ax_h, ay_h, bx_h, by_h, gts_h, out_h,
             av, gv, bcx, bcy, bgi, outv,
             sem_g, sem_a0, sem_a1, sem_a2, sem_a3):
    nc = 2
    wid = lax.axis_index("s") * nc + lax.axis_index("c")
    asems = (sem_a0, sem_a1, sem_a2, sem_a3)
    comps = (ax_h, ay_h, bx_h, by_h)

    cp_g = pltpu.async_copy(gts_h, gv, sem_g)
    copies = []
    for (base, step, cnt, voff), sem in zip(CHUNKS[:3], asems[:3]):
        start = pl.multiple_of(base + step * wid, 16)
        for cc in range(4):
            copies.append(pltpu.async_copy(
                comps[cc].at[pl.ds(start, cnt)],
                av.at[pl.ds(688 * cc + voff, cnt)], sem))
    small = wid < 20

    @pl.when(small)
    def _():
        start = pl.multiple_of(21504 + 16 * wid, 16)
        hs = [pltpu.async_copy(comps[cc].at[pl.ds(start, 16)],
                               av.at[pl.ds(688 * cc + 672, 16)], sem_a3)
              for cc in range(4)]
        for h in hs:
            h.wait()

    sent_vec = jnp.full((L,), SENT, jnp.float32)
    iota = lax.iota(jnp.int32, L)
    cp_g.wait()

    def prep(k, cnts):
        o = k * L
        x0 = gv[pl.ds(0 * GP + o, L)]
        y0 = gv[pl.ds(1 * GP + o, L)]
        x1 = gv[pl.ds(2 * GP + o, L)]
        y1 = gv[pl.ds(3 * GP + o, L)]
        area = (x1 - x0) * (y1 - y0)
        lv = jnp.zeros((L,), jnp.float32)
        for thr in (1024.0, 4096.0, 16384.0, 65536.0):
            lv = lv + jnp.where(area >= thr, 1.0, 0.0).astype(jnp.float32)
        lv = jnp.where(area >= 262144.0, 0.0, lv)
        cx = (x0 + x1) * 0.5
        cy = (y0 + y1) * 0.5
        gidx = iota + o
        valid = gidx < NG
        out = []
        for l in range(5):
            m = (lv == float(l)) & valid
            r = plsc.cumsum(m.astype(jnp.int32))
            dest = cnts[l] + r + (B * l - 1)
            plsc.store_scatter(bcx, [dest], cx, mask=m)
            plsc.store_scatter(bcy, [dest], cy, mask=m)
            plsc.store_scatter(bgi, [dest], gidx, mask=m)
            out.append(cnts[l] + plsc.all_reduce_population_count(m))
        return tuple(out)

    cnts = lax.fori_loop(0, GP // L, prep,
                         tuple(jnp.zeros((L,), jnp.int32) for _ in range(5)))
    c = [jnp.max(cnts[l]) for l in range(5)]
    for l in range(5):
        plsc.store_scatter(bcx, [iota + (B * l + c[l])], sent_vec)

    for handle in copies:
        handle.wait()

    neg2 = jnp.full((L,), -2, jnp.int32)

    def scan_streams(starts, nq, boxes):
        @plsc.parallel_loop(0, nq, carry=(neg2, neg2, neg2, neg2))
        def scan(q, assigns):
            out = list(assigns)
            for u in range(4):
                idx = jnp.full((L,), starts[u] + q, jnp.int32)
                bxv = plsc.load_gather(bcx, [idx])
                byv = plsc.load_gather(bcy, [idx])
                bgv = plsc.load_gather(bgi, [idx])
                a0, a1, a2, a3 = boxes[u]
                m = (bxv > a0) & (byv > a1) & (bxv < a2) & (byv < a3)
                out[u] = jnp.where(m, bgv, out[u])
            return tuple(out)

        return scan

    def load_box(row):
        return [plsc.load_gather(av, [row + 688 * cc]) for cc in range(4)]

    def group(g, carry):
        rows = [iota + (64 * g + 16 * u) for u in range(4)]
        bb = jnp.where(g < 8, 0, B)
        n = jnp.where(g < 8, c[0], c[1])

        @pl.when(n == 0)
        def _():
            for u in range(4):
                plsc.store_scatter(outv, [rows[u]], neg2)

        @pl.when(n > 0)
        def _():
            boxes = [load_box(rows[u]) for u in range(4)]
            nq = (n + 3) >> 2

            @plsc.parallel_loop(0, nq, carry=(neg2, neg2, neg2, neg2))
            def scan(q, assigns):
                out = list(assigns)
                j = bb + q * 4
                for u in range(4):
                    idx = jnp.full((L,), j + u, jnp.int32)
                    bxv = plsc.load_gather(bcx, [idx])
                    byv = plsc.load_gather(bcy, [idx])
                    bgv = plsc.load_gather(bgi, [idx])
                    for si, (a0, a1, a2, a3) in enumerate(boxes):
                        m = (bxv > a0) & (byv > a1) & (bxv < a2) & (byv < a3)
                        out[si] = jnp.where(m, bgv, out[si])
                return tuple(out)

            for u in range(4):
                plsc.store_scatter(outv, [rows[u]], scan[u])
        return carry

    lax.fori_loop(0, 10, group, 0)

    r40 = iota + 640
    r41 = iota + 656
    h2 = (c[2] + 1) >> 1
    b40 = load_box(r40)
    b41 = load_box(r41)
    p = scan_streams([2 * B, 2 * B, 2 * B + h2, 2 * B + h2], h2,
                     [b40, b41, b40, b41])
    plsc.store_scatter(outv, [r40], jnp.maximum(p[0], p[2]))
    plsc.store_scatter(outv, [r41], jnp.maximum(p[1], p[3]))

    r42 = iota + 672
    bb42 = jnp.where(wid < 16, 3 * B, 4 * B)
    n42 = jnp.where(small, jnp.where(wid < 16, c[3], c[4]), 0)
    q4 = (n42 + 3) >> 2
    b42 = load_box(r42)
    p = scan_streams([bb42, bb42 + q4, bb42 + 2 * q4, bb42 + 3 * q4], q4,
                     [b42, b42, b42, b42])
    plsc.store_scatter(outv, [r42],
                       jnp.maximum(jnp.maximum(p[0], p[1]),
                                   jnp.maximum(p[2], p[3])))

    o0 = pltpu.async_copy(outv.at[pl.ds(0, 512)],
                          out_h.at[pl.ds(pl.multiple_of(512 * wid, 16), 512)],
                          sem_a0)
    o1 = pltpu.async_copy(outv.at[pl.ds(512, 128)],
                          out_h.at[pl.ds(pl.multiple_of(16384 + 128 * wid, 16), 128)],
                          sem_a1)
    o2 = pltpu.async_copy(outv.at[pl.ds(640, 32)],
                          out_h.at[pl.ds(pl.multiple_of(20480 + 32 * wid, 16), 32)],
                          sem_a2)

    @pl.when(small)
    def _():
        pltpu.async_copy(outv.at[pl.ds(672, 16)],
                         out_h.at[pl.ds(pl.multiple_of(21504 + 16 * wid, 16), 16)],
                         sem_a3).wait()

    o0.wait()
    o1.wait()
    o2.wait()


@jax.jit
def kernel(anchor, gts):
    cols = [anchor[:, cc] for cc in range(4)]
    gflat = jnp.pad(gts.T, ((0, 0), (0, GP - NG)),
                    constant_values=SENT).reshape(-1)

    mesh = plsc.VectorSubcoreMesh(core_axis_name="c", subcore_axis_name="s")
    run = pl.kernel(
        _sc_body,
        mesh=mesh,
        compiler_params=pltpu.CompilerParams(needs_layout_passes=False,
                                             skip_device_barrier=True),
        out_type=jax.ShapeDtypeStruct((N,), jnp.int32),
        scratch_types=[
            pltpu.VMEM((4 * PER_TILE,), jnp.float32),
            pltpu.VMEM((4 * GP,), jnp.float32),
            pltpu.VMEM((5 * B,), jnp.float32),
            pltpu.VMEM((5 * B,), jnp.float32),
            pltpu.VMEM((5 * B,), jnp.int32),
            pltpu.VMEM((PER_TILE,), jnp.int32),
            pltpu.SemaphoreType.DMA,
            pltpu.SemaphoreType.DMA,
            pltpu.SemaphoreType.DMA,
            pltpu.SemaphoreType.DMA,
            pltpu.SemaphoreType.DMA,
        ],
    )
    return run(*cols, gflat).astype(jnp.int64)


# speedup vs baseline: 1.6371x; 1.0035x over previous
import jax
import jax.numpy as jnp
from jax import lax
from jax.experimental import pallas as pl
from jax.experimental.pallas import tpu as pltpu
from jax.experimental.pallas import tpu_sc as plsc

L = 16
NW = 32
N = 21824
NG = 200
GP = 208
B = 224
PER_TILE = 688
SENT = 2.0e9
NGRP = 12

CHUNKS = ((0, 512, 512, 0), (16384, 128, 128, 512),
          (20480, 32, 32, 640), (21504, 16, 16, 672))


def _sc_body(ax_h, ay_h, bx_h, by_h, gts_h, out_h,
             av, gv, bcx, bcy, bgi, outv,
             sem_g, sem_a0, sem_a1, sem_a2, sem_a3):
    nc = 2
    wid = lax.axis_index("s") * nc + lax.axis_index("c")
    asems = (sem_a0, sem_a1, sem_a2, sem_a3)
    comps = (ax_h, ay_h, bx_h, by_h)

    cp_g = pltpu.async_copy(gts_h, gv, sem_g)
    copies = []
    for (base, step, cnt, voff), sem in zip(CHUNKS[:3], asems[:3]):
        start = pl.multiple_of(base + step * wid, 16)
        for cc in range(4):
            copies.append(pltpu.async_copy(
                comps[cc].at[pl.ds(start, cnt)],
                av.at[pl.ds(688 * cc + voff, cnt)], sem))
    small = wid < 20

    @pl.when(small)
    def _():
        start = pl.multiple_of(21504 + 16 * wid, 16)
        hs = [pltpu.async_copy(comps[cc].at[pl.ds(start, 16)],
                               av.at[pl.ds(688 * cc + 672, 16)], sem_a3)
              for cc in range(4)]
        for h in hs:
            h.wait()

    sent_vec = jnp.full((L,), SENT, jnp.float32)
    iota = lax.iota(jnp.int32, L)
    cp_g.wait()

    def prep(k, cnts):
        o = k * L
        x0 = gv[pl.ds(0 * GP + o, L)]
        y0 = gv[pl.ds(1 * GP + o, L)]
        x1 = gv[pl.ds(2 * GP + o, L)]
        y1 = gv[pl.ds(3 * GP + o, L)]
        area = (x1 - x0) * (y1 - y0)
        lv = jnp.zeros((L,), jnp.float32)
        for thr in (1024.0, 4096.0, 16384.0, 65536.0):
            lv = lv + jnp.where(area >= thr, 1.0, 0.0).astype(jnp.float32)
        lv = jnp.where(area >= 262144.0, 0.0, lv)
        cx = (x0 + x1) * 0.5
        cy = (y0 + y1) * 0.5
        gidx = iota + o
        valid = gidx < NG
        out = []
        for l in range(5):
            m = (lv == float(l)) & valid
            r = plsc.cumsum(m.astype(jnp.int32))
            dest = cnts[l] + r + (B * l - 1)
            plsc.store_scatter(bcx, [dest], cx, mask=m)
            plsc.store_scatter(bcy, [dest], cy, mask=m)
            plsc.store_scatter(bgi, [dest], gidx, mask=m)
            out.append(cnts[l] + plsc.all_reduce_population_count(m))
        return tuple(out)

    cnts = lax.fori_loop(0, GP // L, prep,
                         tuple(jnp.zeros((L,), jnp.int32) for _ in range(5)))
    c = [jnp.max(cnts[l]) for l in range(5)]
    for l in range(5):
        plsc.store_scatter(bcx, [iota + (B * l + c[l])], sent_vec)

    for handle in copies:
        handle.wait()

    neg2 = jnp.full((L,), -2, jnp.int32)

    def scan_streams(starts, nq, boxes):
        @plsc.parallel_loop(0, nq, carry=(neg2, neg2, neg2, neg2))
        def scan(q, assigns):
            out = list(assigns)
            for u in range(4):
                idx = jnp.full((L,), starts[u] + q, jnp.int32)
                bxv = plsc.load_gather(bcx, [idx])
                byv = plsc.load_gather(bcy, [idx])
                bgv = plsc.load_gather(bgi, [idx])
                a0, a1, a2, a3 = boxes[u]
                m = (bxv > a0) & (byv > a1) & (bxv < a2) & (byv < a3)
                out[u] = jnp.where(m, bgv, out[u])
            return tuple(out)

        return scan

    def load_box(row):
        return [plsc.load_gather(av, [row + 688 * cc]) for cc in range(4)]

    def group(g, carry):
        rows = [iota + (64 * g + 16 * u) for u in range(4)]
        bb = jnp.where(g < 8, 0, B)
        n = jnp.where(g < 8, c[0], c[1])

        @pl.when(n == 0)
        def _():
            for u in range(4):
                plsc.store_scatter(outv, [rows[u]], neg2)

        @pl.when(n > 0)
        def _():
            boxes = [load_box(rows[u]) for u in range(4)]
            nq = (n + 3) >> 2

            @plsc.parallel_loop(0, nq, carry=(neg2, neg2, neg2, neg2))
            def scan(q, assigns):
                out = list(assigns)
                j = bb + q * 4
                for u in range(4):
                    idx = jnp.full((L,), j + u, jnp.int32)
                    bxv = plsc.load_gather(bcx, [idx])
                    byv = plsc.load_gather(bcy, [idx])
                    bgv = plsc.load_gather(bgi, [idx])
                    for si, (a0, a1, a2, a3) in enumerate(boxes):
                        m = (bxv > a0) & (byv > a1) & (bxv < a2) & (byv < a3)
                        out[si] = jnp.where(m, bgv, out[si])
                return tuple(out)

            for u in range(4):
                plsc.store_scatter(outv, [rows[u]], scan[u])
        return carry

    lax.fori_loop(0, 10, group, 0)

    o0 = pltpu.async_copy(outv.at[pl.ds(0, 512)],
                          out_h.at[pl.ds(pl.multiple_of(512 * wid, 16), 512)],
                          sem_a0)
    o1 = pltpu.async_copy(outv.at[pl.ds(512, 128)],
                          out_h.at[pl.ds(pl.multiple_of(16384 + 128 * wid, 16), 128)],
                          sem_a1)

    r40 = iota + 640
    r41 = iota + 656
    h2 = (c[2] + 1) >> 1
    b40 = load_box(r40)
    b41 = load_box(r41)
    p = scan_streams([2 * B, 2 * B, 2 * B + h2, 2 * B + h2], h2,
                     [b40, b41, b40, b41])
    plsc.store_scatter(outv, [r40], jnp.maximum(p[0], p[2]))
    plsc.store_scatter(outv, [r41], jnp.maximum(p[1], p[3]))

    r42 = iota + 672
    bb42 = jnp.where(wid < 16, 3 * B, 4 * B)
    n42 = jnp.where(small, jnp.where(wid < 16, c[3], c[4]), 0)
    q4 = (n42 + 3) >> 2
    b42 = load_box(r42)
    p = scan_streams([bb42, bb42 + q4, bb42 + 2 * q4, bb42 + 3 * q4], q4,
                     [b42, b42, b42, b42])
    plsc.store_scatter(outv, [r42],
                       jnp.maximum(jnp.maximum(p[0], p[1]),
                                   jnp.maximum(p[2], p[3])))

    o2 = pltpu.async_copy(outv.at[pl.ds(640, 32)],
                          out_h.at[pl.ds(pl.multiple_of(20480 + 32 * wid, 16), 32)],
                          sem_a2)

    @pl.when(small)
    def _():
        pltpu.async_copy(outv.at[pl.ds(672, 16)],
                         out_h.at[pl.ds(pl.multiple_of(21504 + 16 * wid, 16), 16)],
                         sem_a3).wait()

    o0.wait()
    o1.wait()
    o2.wait()


@jax.jit
def kernel(anchor, gts):
    cols = [anchor[:, cc] for cc in range(4)]
    gflat = jnp.pad(gts.T, ((0, 0), (0, GP - NG)),
                    constant_values=SENT).reshape(-1)

    mesh = plsc.VectorSubcoreMesh(core_axis_name="c", subcore_axis_name="s")
    run = pl.kernel(
        _sc_body,
        mesh=mesh,
        compiler_params=pltpu.CompilerParams(needs_layout_passes=False,
                                             skip_device_barrier=True),
        out_type=jax.ShapeDtypeStruct((N,), jnp.int32),
        scratch_types=[
            pltpu.VMEM((4 * PER_TILE,), jnp.float32),
            pltpu.VMEM((4 * GP,), jnp.float32),
            pltpu.VMEM((5 * B,), jnp.float32),
            pltpu.VMEM((5 * B,), jnp.float32),
            pltpu.VMEM((5 * B,), jnp.int32),
            pltpu.VMEM((PER_TILE,), jnp.int32),
            pltpu.SemaphoreType.DMA,
            pltpu.SemaphoreType.DMA,
            pltpu.SemaphoreType.DMA,
            pltpu.SemaphoreType.DMA,
            pltpu.SemaphoreType.DMA,
        ],
    )
    return run(*cols, gflat).astype(jnp.int64)
